# baseline (speedup 1.0000x reference)
import functools

import jax
import jax.numpy as jnp
from jax import lax
from jax.experimental import pallas as pl
from jax.experimental.pallas import tpu as pltpu
from jax.experimental.pallas import tpu_sc as plsc

B = 16384
D = 128
H = 512
OUT = 3

_info = plsc.get_sparse_core_info()
_NC, _NS = _info.num_cores, _info.num_subcores
_NW = _NC * _NS


def _sc_gather_pack(table, idx2d, nb):
    ch = (nb // 128) // _NW
    mesh = plsc.VectorSubcoreMesh(core_axis_name="c", subcore_axis_name="s")

    @functools.partial(
        pl.kernel,
        mesh=mesh,
        out_type=jax.ShapeDtypeStruct((nb // 2, D), jnp.int32),
        scratch_types=[
            pltpu.VMEM((ch, 128), jnp.int32),
            pltpu.VMEM((ch, 128, D), jnp.int32),
            pltpu.VMEM((ch, 64, D), jnp.int32),
            pltpu.SemaphoreType.DMA,
            pltpu.SemaphoreType.DMA,
        ],
    )
    def k(table_hbm, idx_hbm, out_hbm, idx_v, rows_v, xp_v, gsem, osem):
        wid = lax.axis_index("s") * _NC + lax.axis_index("c")
        base = wid * ch
        pltpu.sync_copy(idx_hbm.at[pl.ds(base, ch)], idx_v)
        gathers = [
            pltpu.async_copy(table_hbm.at[idx_v.at[j]], rows_v.at[j], gsem)
            for j in range(ch)
        ]
        out_copies = []
        for j in range(ch):
            gathers[j].wait()

            def conv(kk, _, j=j):
                r = kk + kk
                for c in range(D // 16):
                    a = rows_v[j, r, pl.ds(c * 16, 16)]
                    b = rows_v[j, r + 1, pl.ds(c * 16, 16)]
                    ra = a + 0x7FFF + (lax.shift_right_logical(a, 16) & 1)
                    rb = b + 0x7FFF + (lax.shift_right_logical(b, 16) & 1)
                    xp_v[j, kk, pl.ds(c * 16, 16)] = lax.shift_right_logical(
                        ra, 16
                    ) | (rb & jnp.int32(-65536))
                return 0

            lax.fori_loop(0, 64, conv, 0)
            out_copies.append(
                pltpu.async_copy(
                    xp_v.at[j], out_hbm.at[pl.ds((base + j) * 64, 64)], osem
                )
            )
        for c in out_copies:
            c.wait()

    return k(table, idx2d)


def _tc_mlp(xp, w1tb, b1r, w2, b2c, nb):
    blk = 4096

    def body(x_ref, w1_ref, b1_ref, w2_ref, b2_ref, o_ref):
        xb = pltpu.bitcast(x_ref[:], jnp.bfloat16)
        h = jnp.dot(xb, w1_ref[:], preferred_element_type=jnp.float32)
        h = jnp.maximum(h + b1_ref[:], 0.0)
        ot = lax.dot_general(
            w2_ref[:], h, (((1,), (1,)), ((), ())),
            preferred_element_type=jnp.float32,
        )
        o_ref[:] = ot + b2_ref[:]

    return pl.pallas_call(
        body,
        grid=(nb // blk,),
        in_specs=[
            pl.BlockSpec((blk // 2, D), lambda i: (i, 0)),
            pl.BlockSpec((D, H), lambda i: (0, 0)),
            pl.BlockSpec((1, H), lambda i: (0, 0)),
            pl.BlockSpec((OUT, H), lambda i: (0, 0)),
            pl.BlockSpec((OUT, 1), lambda i: (0, 0)),
        ],
        out_specs=pl.BlockSpec((OUT, blk), lambda i: (0, i)),
        out_shape=jax.ShapeDtypeStruct((OUT, nb), jnp.float32),
    )(xp, w1tb, b1r, w2, b2c)


def kernel(geo_id, emb_table, W1, b1, W2, b2):
    idx2d = geo_id.astype(jnp.int32).reshape(B // 128, 128)
    table_i32 = lax.bitcast_convert_type(emb_table, jnp.int32)
    xp = _sc_gather_pack(table_i32, idx2d, B)
    w1tb = W1.T.astype(jnp.bfloat16)
    out_t = _tc_mlp(xp, w1tb, b1.reshape(1, H), W2, b2.reshape(OUT, 1), B)
    return out_t.T


# speedup vs baseline: 1.7109x; 1.7109x over previous
import functools

import jax
import jax.numpy as jnp
from jax import lax
from jax.experimental import pallas as pl
from jax.experimental.pallas import tpu as pltpu
from jax.experimental.pallas import tpu_sc as plsc

B = 16384
D = 128
H = 512
OUT = 3

_info = plsc.get_sparse_core_info()
_NC, _NS = _info.num_cores, _info.num_subcores
_NW = _NC * _NS


def _sc_gather_pack(table, idx2d, nb):
    ch = (nb // 128) // _NW
    mesh = plsc.VectorSubcoreMesh(core_axis_name="c", subcore_axis_name="s")

    @functools.partial(
        pl.kernel,
        mesh=mesh,
        out_type=jax.ShapeDtypeStruct((nb // 2, D), jnp.int32),
        scratch_types=[
            pltpu.VMEM((ch, 128), jnp.int32),
            pltpu.VMEM((ch, 128, D), jnp.float32),
            pltpu.VMEM((ch, 64, D), jnp.int32),
            pltpu.SemaphoreType.DMA,
            pltpu.SemaphoreType.DMA,
        ],
    )
    def k(table_hbm, idx_hbm, out_hbm, idx_v, rows_v, xp_v, gsem, osem):
        wid = lax.axis_index("s") * _NC + lax.axis_index("c")
        base = wid * ch
        pltpu.sync_copy(idx_hbm.at[pl.ds(base, ch)], idx_v)
        gathers = [
            pltpu.async_copy(table_hbm.at[idx_v.at[j]], rows_v.at[j], gsem)
            for j in range(ch)
        ]
        out_copies = []
        for j in range(ch):
            gathers[j].wait()

            rows_i = rows_v.bitcast(jnp.int32)

            def conv(kk, _, j=j):
                r = kk + kk
                for c in range(D // 16):
                    a = rows_i[j, r, pl.ds(c * 16, 16)]
                    b = rows_i[j, r + 1, pl.ds(c * 16, 16)]
                    xp_v[j, kk, pl.ds(c * 16, 16)] = lax.shift_right_logical(
                        a, 16
                    ) | (b & jnp.int32(-65536))
                return 0

            lax.fori_loop(0, 64, conv, 0)
            out_copies.append(
                pltpu.async_copy(
                    xp_v.at[j], out_hbm.at[pl.ds((base + j) * 64, 64)], osem
                )
            )
        for c in out_copies:
            c.wait()

    return k(table, idx2d)


def _tc_mlp(xp, w1tb, b1r, w2, b2c, nb):
    blk = 4096

    def body(x_ref, w1_ref, b1_ref, w2_ref, b2_ref, o_ref):
        xb = pltpu.bitcast(x_ref[:], jnp.bfloat16)
        h = jnp.dot(xb, w1_ref[:], preferred_element_type=jnp.float32)
        h = jnp.maximum(h + b1_ref[:], 0.0)
        ot = lax.dot_general(
            w2_ref[:], h, (((1,), (1,)), ((), ())),
            preferred_element_type=jnp.float32,
        )
        o_ref[:] = ot + b2_ref[:]

    return pl.pallas_call(
        body,
        grid=(nb // blk,),
        in_specs=[
            pl.BlockSpec((blk // 2, D), lambda i: (i, 0)),
            pl.BlockSpec((D, H), lambda i: (0, 0)),
            pl.BlockSpec((1, H), lambda i: (0, 0)),
            pl.BlockSpec((OUT, H), lambda i: (0, 0)),
            pl.BlockSpec((OUT, 1), lambda i: (0, 0)),
        ],
        out_specs=pl.BlockSpec((OUT, blk), lambda i: (0, i)),
        out_shape=jax.ShapeDtypeStruct((OUT, nb), jnp.float32),
    )(xp, w1tb, b1r, w2, b2c)


def kernel(geo_id, emb_table, W1, b1, W2, b2):
    idx2d = geo_id.astype(jnp.int32).reshape(B // 128, 128)
    xp = _sc_gather_pack(emb_table, idx2d, B)
    w1tb = W1.T.astype(jnp.bfloat16)
    out_t = _tc_mlp(xp, w1tb, b1.reshape(1, H), W2, b2.reshape(OUT, 1), B)
    return out_t.T
